# Initial kernel scaffold; baseline (speedup 1.0000x reference)
import jax
import jax.numpy as jnp
from jax.experimental import pallas as pl


def kernel(x, undirected_edges_small, directed_edges_small, undirected_edges_small_middle, undirected_edges_middle, directed_edges_middle, subgraph_edges, W1, a1s, a1d, b1, W2, a2s, a2d, b2, W3, a3s, a3d, b3, W4, a4s, a4d, b4, W5, a5s, a5d, b5, Wfc1, bfc1, Wfc2, bfc2, Wfc3, bfc3, Wfc4, bfc4, Wreg, breg):
    raise NotImplementedError("write your pallas kernel here")


# speedup vs baseline: 2.9905x; 2.9905x over previous
import functools

import jax
import jax.numpy as jnp
from jax.experimental import pallas as pl

_H = 2


def _mm_kernel(x_ref, w_ref, b_ref, o_ref, *, nk, relu):
    @pl.when(pl.program_id(2) == 0)
    def _init():
        o_ref[...] = jnp.zeros_like(o_ref)

    o_ref[...] += jnp.dot(x_ref[...], w_ref[...],
                          preferred_element_type=jnp.float32)

    @pl.when(pl.program_id(2) == nk - 1)
    def _fin():
        r = o_ref[...] + b_ref[...]
        if relu:
            r = jnp.maximum(r, 0.0)
        o_ref[...] = r


def _matmul_bias(x, w, b, relu=False, bm=256, bn=512):
    n, k = x.shape
    m = w.shape[1]
    bm = min(bm, n)
    bn = min(bn, m)
    bk = min(2048, k)
    nk = pl.cdiv(k, bk)
    kp = nk * bk
    if kp != k:
        x = jnp.pad(x, ((0, 0), (0, kp - k)))
        w = jnp.pad(w, ((0, kp - k), (0, 0)))
    b2 = b.reshape(1, m)
    grid = (pl.cdiv(n, bm), pl.cdiv(m, bn), nk)
    return pl.pallas_call(
        functools.partial(_mm_kernel, nk=nk, relu=relu),
        grid=grid,
        in_specs=[
            pl.BlockSpec((bm, bk), lambda i, j, kk: (i, kk)),
            pl.BlockSpec((bk, bn), lambda i, j, kk: (kk, j)),
            pl.BlockSpec((1, bn), lambda i, j, kk: (0, j)),
        ],
        out_specs=pl.BlockSpec((bm, bn), lambda i, j, kk: (i, j)),
        out_shape=jax.ShapeDtypeStruct((n, m), jnp.float32),
    )(x, w, b2)


def _leaky_kernel(a_ref, b_ref, o_ref):
    s = a_ref[...] + b_ref[...]
    o_ref[...] = jnp.where(s > 0, s, 0.2 * s)


def _expsub_kernel(e_ref, m_ref, o_ref):
    o_ref[...] = jnp.exp(e_ref[...] - m_ref[...])


def _div_kernel(x_ref, d_ref, o_ref):
    o_ref[...] = x_ref[...] / (d_ref[...] + 1e-16)


def _scale_kernel(x_ref, a_ref, o_ref):
    o_ref[...] = x_ref[...] * a_ref[...]


def _elemwise(kfn, args, out_cols):
    rows = args[0].shape[0]
    width = max([out_cols] + [a.shape[1] for a in args])
    br = min(rows, max(8, (1 << 18) // width), 8192)
    grid = (pl.cdiv(rows, br),)
    specs = [pl.BlockSpec((br, a.shape[1]), lambda i: (i, 0)) for a in args]
    return pl.pallas_call(
        kfn,
        grid=grid,
        in_specs=specs,
        out_specs=pl.BlockSpec((br, out_cols), lambda i: (i, 0)),
        out_shape=jax.ShapeDtypeStruct((rows, out_cols), jnp.float32),
    )(*args)


def _biasrelu_kernel(x_ref, b_ref, o_ref, *, relu):
    r = x_ref[...] + b_ref[...]
    if relu:
        r = jnp.maximum(r, 0.0)
    o_ref[...] = r


def _bias_relu(x, b, relu):
    rows, cols = x.shape
    br = min(rows, max(8, (1 << 18) // cols))
    return pl.pallas_call(
        functools.partial(_biasrelu_kernel, relu=relu),
        grid=(pl.cdiv(rows, br),),
        in_specs=[pl.BlockSpec((br, cols), lambda i: (i, 0)),
                  pl.BlockSpec((1, cols), lambda i: (0, 0))],
        out_specs=pl.BlockSpec((br, cols), lambda i: (i, 0)),
        out_shape=jax.ShapeDtypeStruct((rows, cols), jnp.float32),
    )(x, b.reshape(1, cols))


def _gat(x, edge_index, W, att_s, att_d, bias, relu_out):
    N = x.shape[0]
    out_ch = W.shape[1] // _H
    loops = jnp.arange(N, dtype=edge_index.dtype)
    src = jnp.concatenate([edge_index[0], loops])
    dst = jnp.concatenate([edge_index[1], loops])
    E = src.shape[0]

    xp = _matmul_bias(x, W, jnp.zeros((W.shape[1],), jnp.float32))

    proj = jnp.zeros((_H * out_ch, 2 * _H), jnp.float32)
    for h in range(_H):
        proj = proj.at[h * out_ch:(h + 1) * out_ch, h].set(att_s[h])
        proj = proj.at[h * out_ch:(h + 1) * out_ch, _H + h].set(att_d[h])
    ab = _matmul_bias(xp, proj, jnp.zeros((2 * _H,), jnp.float32))
    a_s = ab[:, :_H]
    a_d = ab[:, _H:]

    e = _elemwise(_leaky_kernel, [a_s[src], a_d[dst]], _H)
    e_max = jax.ops.segment_max(e, dst, num_segments=N)
    e_max = jnp.where(jnp.isfinite(e_max), e_max, 0.0)
    ex = _elemwise(_expsub_kernel, [e, e_max[dst]], _H)
    denom = jax.ops.segment_sum(ex, dst, num_segments=N)
    alpha = _elemwise(_div_kernel, [ex, denom[dst]], _H)

    xg = xp[src].reshape(E * _H, out_ch)
    msg = _elemwise(_scale_kernel, [xg, alpha.reshape(E * _H, 1)],
                    out_ch).reshape(E, _H * out_ch)
    out = jax.ops.segment_sum(msg, dst, num_segments=N)
    return _bias_relu(out, bias, relu_out)


def kernel(x, undirected_edges_small, directed_edges_small,
           undirected_edges_small_middle, undirected_edges_middle,
           directed_edges_middle, subgraph_edges,
           W1, a1s, a1d, b1, W2, a2s, a2d, b2, W3, a3s, a3d, b3,
           W4, a4s, a4d, b4, W5, a5s, a5d, b5,
           Wfc1, bfc1, Wfc2, bfc2, Wfc3, bfc3, Wfc4, bfc4, Wreg, breg):
    N = x.shape[0]
    h = _gat(x, undirected_edges_small, W1, a1s, a1d, b1, False)
    h = _gat(h, directed_edges_small, W2, a2s, a2d, b2, True)
    h = _gat(h, undirected_edges_small_middle, W3, a3s, a3d, b3, True)
    h = _gat(h, undirected_edges_middle, W4, a4s, a4d, b4, False)
    h = _gat(h, directed_edges_middle, W5, a5s, a5d, b5, True)

    S = subgraph_edges.shape[0]
    flat = subgraph_edges.reshape(S, -1)
    present = jnp.zeros((S, N), jnp.float32).at[
        jnp.arange(S)[:, None], flat].set(1.0)
    counts = present.sum(axis=1)
    g = _matmul_bias(present, h, jnp.zeros((h.shape[1],), jnp.float32))
    g = g / counts[:, None]

    g = _matmul_bias(g, Wfc1, bfc1, relu=True)
    g = _matmul_bias(g, Wfc2, bfc2, relu=True)
    g = _matmul_bias(g, Wfc3, bfc3, relu=True)
    g = _matmul_bias(g, Wfc4, bfc4, relu=True)
    return _matmul_bias(g, Wreg, breg)
